# baseline (speedup 1.0000x reference)
import functools

import jax
import jax.numpy as jnp
from jax import lax
from jax.experimental import pallas as pl
from jax.experimental.pallas import tpu as pltpu
from jax.experimental.pallas import tpu_sc as plsc

_B = 16384
_E = 25
_VOCAB = 26
_PAD_ROW = 26
_VP = 32
_BSC = 8192
_BTC = _B - _BSC
_NS = 16
_BPW = _BSC // _NS
_L = 16
_TBLK = 1024

_GDN = lax.GatherDimensionNumbers(
    offset_dims=(), collapsed_slice_dims=(0,), start_index_map=(0,)
)


def _vreg_gather(vec, idx):
    return lax.gather(
        vec, idx[:, None], _GDN, (1,),
        mode=lax.GatherScatterMode.PROMISE_IN_BOUNDS,
    )


@functools.lru_cache(maxsize=1)
def _build_sc():
    mesh = plsc.VectorSubcoreMesh(
        core_axis_name="c", subcore_axis_name="s",
        num_cores=1, num_subcores=_NS,
    )

    @functools.partial(
        pl.kernel,
        out_type=jax.ShapeDtypeStruct((_E, _BSC), jnp.float32),
        mesh=mesh,
        scratch_types=[
            pltpu.VMEM((_E, _VP), jnp.float32),
            pltpu.VMEM((_BPW,), jnp.int32),
            pltpu.VMEM((_BPW,), jnp.int32),
            pltpu.VMEM((_E, _BPW), jnp.float32),
            pltpu.SemaphoreType.DMA,
        ],
        compiler_params=pltpu.CompilerParams(needs_layout_passes=False),
    )
    def emb(wt_hbm, col_hbm, len_hbm, out_hbm, tab_v, col_v, len_v, outt_v, sem):
        wid = lax.axis_index("s")
        base = wid * _BPW
        cps = [
            pltpu.async_copy(wt_hbm, tab_v, sem),
            pltpu.async_copy(col_hbm.at[pl.ds(base, _BPW)], col_v, sem),
            pltpu.async_copy(len_hbm.at[pl.ds(base, _BPW)], len_v, sem),
        ]
        for cp in cps:
            cp.wait()

        @plsc.parallel_loop(0, _BPW, _L, unroll=1)
        def _(off):
            tok = col_v[pl.ds(off, _L)]
            ln = len_v[pl.ds(off, _L)]
            idx = jnp.where(ln > 0, tok, _PAD_ROW)
            lo = idx < _L
            idxm = lax.bitwise_and(idx, _L - 1)
            for c in range(_E):
                va = _vreg_gather(tab_v[c, pl.ds(0, _L)], idxm)
                vb = _vreg_gather(tab_v[c, pl.ds(_L, _L)], idxm)
                outt_v[c, pl.ds(off, _L)] = jnp.where(lo, va, vb)

        pltpu.sync_copy(outt_v, out_hbm.at[:, pl.ds(base, _BPW)])

    return emb


def _tc_body(col_ref, len_ref, wt_ref, out_ref):
    tok = col_ref[...]
    ln = len_ref[...]
    vocab = lax.broadcasted_iota(jnp.int32, (_TBLK, _VP), 1)
    oh = jnp.where(
        (tok[:, None] == vocab) & (ln[:, None] > 0), 1.0, 0.0
    ).astype(jnp.float32)
    out_ref[...] = lax.dot_general(
        oh, wt_ref[...], (((1,), (1,)), ((), ())),
        preferred_element_type=jnp.float32,
    )


@functools.lru_cache(maxsize=1)
def _build_tc():
    return pl.pallas_call(
        _tc_body,
        grid=(_BTC // _TBLK,),
        in_specs=[
            pl.BlockSpec((_TBLK,), lambda i: (i,)),
            pl.BlockSpec((_TBLK,), lambda i: (i,)),
            pl.BlockSpec((_E, _VP), lambda i: (0, 0)),
        ],
        out_specs=pl.BlockSpec((_TBLK, _E), lambda i: (i, 0)),
        out_shape=jax.ShapeDtypeStruct((_BTC, _E), jnp.float32),
    )


def kernel(input, lengths, weight):
    col = input[:, 0]
    wt = jnp.pad(weight.T, ((0, 0), (0, _VP - _VOCAB)))
    outt_sc = _build_sc()(wt, col[:_BSC], lengths[:_BSC])
    out_tc = _build_tc()(col[_BSC:], lengths[_BSC:], wt)
    return jnp.concatenate([outt_sc.T, out_tc], axis=0)


# speedup vs baseline: 1.3983x; 1.3983x over previous
import functools

import jax
import jax.numpy as jnp
from jax import lax
from jax.experimental import pallas as pl
from jax.experimental.pallas import tpu as pltpu
from jax.experimental.pallas import tpu_sc as plsc

_B = 16384
_E = 25
_VOCAB = 26
_PAD_ROW = 26
_VP = 32
_NC = 1
_NS = 16
_NW = _NC * _NS
_BPW = _B // _NW
_L = 16
_NCHUNK = 4
_CHW = _BPW // _NCHUNK

_GDN = lax.GatherDimensionNumbers(
    offset_dims=(), collapsed_slice_dims=(0,), start_index_map=(0,)
)


def _vreg_gather(vec, idx):
    return lax.gather(
        vec, idx[:, None], _GDN, (1,),
        mode=lax.GatherScatterMode.PROMISE_IN_BOUNDS,
    )


@functools.lru_cache(maxsize=1)
def _build():
    mesh = plsc.VectorSubcoreMesh(
        core_axis_name="c", subcore_axis_name="s",
        num_cores=_NC, num_subcores=_NS,
    )

    @functools.partial(
        pl.kernel,
        out_type=jax.ShapeDtypeStruct((_E, _B), jnp.float32),
        mesh=mesh,
        scratch_types=[
            pltpu.VMEM((_E, _VP), jnp.float32),
            pltpu.VMEM((_BPW,), jnp.int32),
            pltpu.VMEM((_BPW,), jnp.int32),
            pltpu.VMEM((_E, _BPW), jnp.float32),
            pltpu.SemaphoreType.DMA,
        ],
        compiler_params=pltpu.CompilerParams(needs_layout_passes=False),
    )
    def emb(wt_hbm, col_hbm, len_hbm, out_hbm, tab_v, col_v, len_v, outt_v, sem):
        wid = lax.axis_index("s") * _NC + lax.axis_index("c")
        base = wid * _BPW
        cps = [
            pltpu.async_copy(wt_hbm, tab_v, sem),
            pltpu.async_copy(col_hbm.at[pl.ds(base, _BPW)], col_v, sem),
            pltpu.async_copy(len_hbm.at[pl.ds(base, _BPW)], len_v, sem),
        ]
        for cp in cps:
            cp.wait()

        @plsc.parallel_loop(0, _BPW, _L, unroll=1)
        def _(off):
            tok = col_v[pl.ds(off, _L)]
            ln = len_v[pl.ds(off, _L)]
            idx = jnp.where(ln > 0, tok, _PAD_ROW)
            lo = idx < _L
            idxm = lax.bitwise_and(idx, _L - 1)
            for c in range(_E):
                va = _vreg_gather(tab_v[c, pl.ds(0, _L)], idxm)
                vb = _vreg_gather(tab_v[c, pl.ds(_L, _L)], idxm)
                outt_v[c, pl.ds(off, _L)] = jnp.where(lo, va, vb)

        pltpu.sync_copy(outt_v, out_hbm.at[:, pl.ds(base, _BPW)])

    return emb


def kernel(input, lengths, weight):
    col = input[:, 0]
    wt = jnp.pad(weight.T, ((0, 0), (0, _VP - _VOCAB)))
    outt = _build()(wt, col, lengths)
    return outt.T
